# baseline (speedup 1.0000x reference)
import functools

import jax
import jax.numpy as jnp
from jax import lax
from jax.experimental import pallas as pl
from jax.experimental.pallas import tpu as pltpu
from jax.experimental.pallas import tpu_sc as plsc

EDGE_COUNT = 100000
NP = 256
ACC = NP * NP
NC, NS, LANES = 1, 16, 16
NW = NC * NS
EPT = 6400
E_PAD = NW * EPT
NIDX = EPT // 128
ZPT = ACC // NS


def _sc_body(src_hbm, dst_hbm, w_hbm, out_hbm,
             src_v, dst_v, w_v, one_v, idxw_v, idxc_v, stage_v, sem, acc_sh):
    cid = lax.axis_index("c")
    sid = lax.axis_index("s")
    tile = cid * NS + sid

    cp_s = pltpu.async_copy(src_hbm.at[tile], src_v, sem)
    cp_d = pltpu.async_copy(dst_hbm.at[tile], dst_v, sem)
    cp_w = pltpu.async_copy(w_hbm.at[tile], w_v, sem)

    def zloop(i, c):
        stage_v[pl.ds(i * LANES, LANES)] = jnp.zeros((LANES,), jnp.float32)
        return c
    lax.fori_loop(0, ZPT // LANES, zloop, 0)
    pltpu.sync_copy(stage_v, acc_sh.at[pl.ds(sid * ZPT, ZPT)])
    cp_s.wait()
    cp_d.wait()
    cp_w.wait()

    ebase = tile * EPT
    def iloop(r, c):
        off = r * LANES
        s = src_v[pl.ds(off, LANES)]
        d = dst_v[pl.ds(off, LANES)]
        lane = lax.iota(jnp.int32, LANES)
        idxw_v[pl.ds(off, LANES)] = d * NP + s
        idxc_v[pl.ds(off, LANES)] = d * NP + 200 + lane
        eid = ebase + off + lane
        one_v[pl.ds(off, LANES)] = jnp.where(eid < EDGE_COUNT, 1.0, 0.0)
        return c
    lax.fori_loop(0, EPT // LANES, iloop, 0)

    plsc.subcore_barrier()

    pltpu.sync_copy(w_v, acc_sh.at[idxw_v], add=True)
    pltpu.sync_copy(one_v, acc_sh.at[idxc_v], add=True)

    plsc.subcore_barrier()

    pltpu.sync_copy(acc_sh.at[pl.ds(sid * ZPT, ZPT)], stage_v)
    pltpu.sync_copy(stage_v, out_hbm.at[cid, pl.ds(sid * ZPT, ZPT)])


@functools.cache
def _get_build_adj():
    return pl.kernel(
        _sc_body,
        out_type=jax.ShapeDtypeStruct((NC, ACC), jnp.float32),
        mesh=plsc.VectorSubcoreMesh(core_axis_name="c", subcore_axis_name="s",
                                    num_cores=NC, num_subcores=NS),
        scratch_types=[
            pltpu.VMEM((EPT,), jnp.int32),
            pltpu.VMEM((EPT,), jnp.int32),
            pltpu.VMEM((EPT,), jnp.float32),
            pltpu.VMEM((EPT,), jnp.float32),
            pltpu.VMEM((EPT,), jnp.int32),
            pltpu.VMEM((EPT,), jnp.int32),
            pltpu.VMEM((ZPT,), jnp.float32),
            pltpu.SemaphoreType.DMA,
            pltpu.VMEM_SHARED((ACC,), jnp.float32),
        ],
    )


def _tc_body(ap_ref, x_ref, wl1_ref, bl1_ref, wr1_ref, wl2_ref, bl2_ref,
             wr2_ref, w1_ref, b1_ref, w2_ref, b2_ref, w3_ref, b3_ref, o_ref):
    f32 = jnp.float32
    dot = functools.partial(lax.dot_general, preferred_element_type=f32)

    A = ap_ref[0]
    for i in range(1, NC):
        A = A + ap_ref[i]
    lanes = lax.broadcasted_iota(jnp.int32, (NP, NP), 1)
    cnt = jnp.sum(jnp.where(lanes >= 200, A, 0.0), axis=1, keepdims=True)
    inv = 1.0 / jnp.maximum(cnt, 1.0)
    rowmask = (lax.broadcasted_iota(jnp.int32, (NP, 1), 0) < 200).astype(f32)

    x = x_ref[...]
    mean1 = dot(A, x, (((1,), (0,)), ((), ()))) * inv
    h = dot(mean1, wl1_ref[...], (((1,), (1,)), ((), ()))) + bl1_ref[...] \
        + dot(x, wr1_ref[...], (((1,), (1,)), ((), ())))
    h = jnp.maximum(h, 0.0) * rowmask
    mean2 = dot(A, h, (((1,), (0,)), ((), ()))) * inv
    out = dot(mean2, wl2_ref[...], (((1,), (1,)), ((), ()))) + bl2_ref[...] \
        + dot(h, wr2_ref[...], (((1,), (1,)), ((), ())))
    t = dot(w1_ref[...], out, (((1,), (0,)), ((), ()))) + b1_ref[...]
    t = jnp.maximum(t, 0.0)
    t = dot(w2_ref[...], t, (((1,), (0,)), ((), ()))) + b2_ref[...]
    t = jnp.maximum(t, 0.0)
    o_ref[...] = dot(w3_ref[...], t, (((1,), (0,)), ((), ()))) + b3_ref[...]


_dense = pl.pallas_call(
    _tc_body,
    out_shape=jax.ShapeDtypeStruct((10, 64), jnp.float32),
)


def kernel(x, edge_index, edge_attr, W_l1, b_l1, W_r1, W_l2, b_l2, W_r2,
           W1, b1, W2, b2, W3, b3):
    src = edge_index[0].astype(jnp.int32)
    dst = edge_index[1].astype(jnp.int32)
    pad = E_PAD - src.shape[0]
    src_p = jnp.pad(src, (0, pad)).reshape(NW, EPT)
    dst_p = jnp.pad(dst, (0, pad)).reshape(NW, EPT)
    w_p = jnp.pad(edge_attr.astype(jnp.float32), (0, pad)).reshape(NW, EPT)

    a_parts = _get_build_adj()(src_p, dst_p, w_p).reshape(NC, NP, NP)

    x_pad = jnp.pad(x, ((0, NP - x.shape[0]), (0, 0)))
    w1_pad = jnp.pad(W1, ((0, 0), (0, NP - W1.shape[1])))
    out = _dense(a_parts, x_pad, W_l1, b_l1.reshape(1, -1), W_r1,
                 W_l2, b_l2.reshape(1, -1), W_r2,
                 w1_pad, b1.reshape(-1, 1), W2, b2.reshape(-1, 1),
                 W3, b3.reshape(-1, 1))
    return out.T


# speedup vs baseline: 1.1944x; 1.1944x over previous
import functools

import jax
import jax.numpy as jnp
from jax import lax
from jax.experimental import pallas as pl
from jax.experimental.pallas import tpu as pltpu
from jax.experimental.pallas import tpu_sc as plsc

EDGE_COUNT = 100000
NP = 256
ACC = NP * NP
NC, NS, LANES = 2, 16, 16
NW = NC * NS
EPT = 3200
E_PAD = NW * EPT
NIDX = EPT // 128
ZPT = ACC // NS


def _sc_body(src_hbm, dst_hbm, w_hbm, out_hbm,
             src_v, dst_v, w_v, one_v, idxw_v, idxc_v, stage_v, sem, acc_sh):
    cid = lax.axis_index("c")
    sid = lax.axis_index("s")
    tile = cid * NS + sid

    cp_s = pltpu.async_copy(src_hbm.at[tile], src_v, sem)
    cp_d = pltpu.async_copy(dst_hbm.at[tile], dst_v, sem)
    cp_w = pltpu.async_copy(w_hbm.at[tile], w_v, sem)

    def zloop(i, c):
        stage_v[pl.ds(i * LANES, LANES)] = jnp.zeros((LANES,), jnp.float32)
        return c
    lax.fori_loop(0, ZPT // LANES, zloop, 0)
    pltpu.sync_copy(stage_v, acc_sh.at[pl.ds(sid * ZPT, ZPT)])
    cp_s.wait()
    cp_d.wait()
    cp_w.wait()

    ebase = tile * EPT
    def iloop(r, c):
        off = r * LANES
        s = src_v[pl.ds(off, LANES)]
        d = dst_v[pl.ds(off, LANES)]
        lane = lax.iota(jnp.int32, LANES)
        idxw_v[pl.ds(off, LANES)] = d * NP + s
        idxc_v[pl.ds(off, LANES)] = d * NP + 200 + lane
        eid = ebase + off + lane
        one_v[pl.ds(off, LANES)] = jnp.where(eid < EDGE_COUNT, 1.0, 0.0)
        return c
    lax.fori_loop(0, EPT // LANES, iloop, 0)

    plsc.subcore_barrier()

    pltpu.sync_copy(w_v, acc_sh.at[idxw_v], add=True)
    pltpu.sync_copy(one_v, acc_sh.at[idxc_v], add=True)

    plsc.subcore_barrier()

    pltpu.sync_copy(acc_sh.at[pl.ds(sid * ZPT, ZPT)], stage_v)
    pltpu.sync_copy(stage_v, out_hbm.at[cid, pl.ds(sid * ZPT, ZPT)])


@functools.cache
def _get_build_adj():
    return pl.kernel(
        _sc_body,
        out_type=jax.ShapeDtypeStruct((NC, ACC), jnp.float32),
        mesh=plsc.VectorSubcoreMesh(core_axis_name="c", subcore_axis_name="s",
                                    num_cores=NC, num_subcores=NS),
        scratch_types=[
            pltpu.VMEM((EPT,), jnp.int32),
            pltpu.VMEM((EPT,), jnp.int32),
            pltpu.VMEM((EPT,), jnp.float32),
            pltpu.VMEM((EPT,), jnp.float32),
            pltpu.VMEM((EPT,), jnp.int32),
            pltpu.VMEM((EPT,), jnp.int32),
            pltpu.VMEM((ZPT,), jnp.float32),
            pltpu.SemaphoreType.DMA,
            pltpu.VMEM_SHARED((ACC,), jnp.float32),
        ],
    )


def _tc_body(ap_ref, x_ref, wl1_ref, bl1_ref, wr1_ref, wl2_ref, bl2_ref,
             wr2_ref, w1_ref, b1_ref, w2_ref, b2_ref, w3_ref, b3_ref, o_ref):
    f32 = jnp.float32
    dot = functools.partial(lax.dot_general, preferred_element_type=f32)

    A = ap_ref[0]
    for i in range(1, NC):
        A = A + ap_ref[i]
    lanes = lax.broadcasted_iota(jnp.int32, (NP, NP), 1)
    cnt = jnp.sum(jnp.where(lanes >= 200, A, 0.0), axis=1, keepdims=True)
    inv = 1.0 / jnp.maximum(cnt, 1.0)
    rowmask = (lax.broadcasted_iota(jnp.int32, (NP, 1), 0) < 200).astype(f32)

    x = x_ref[...]
    mean1 = dot(A, x, (((1,), (0,)), ((), ()))) * inv
    h = dot(mean1, wl1_ref[...], (((1,), (1,)), ((), ()))) + bl1_ref[...] \
        + dot(x, wr1_ref[...], (((1,), (1,)), ((), ())))
    h = jnp.maximum(h, 0.0) * rowmask
    mean2 = dot(A, h, (((1,), (0,)), ((), ()))) * inv
    out = dot(mean2, wl2_ref[...], (((1,), (1,)), ((), ()))) + bl2_ref[...] \
        + dot(h, wr2_ref[...], (((1,), (1,)), ((), ())))
    t = dot(w1_ref[...], out, (((1,), (0,)), ((), ()))) + b1_ref[...]
    t = jnp.maximum(t, 0.0)
    t = dot(w2_ref[...], t, (((1,), (0,)), ((), ()))) + b2_ref[...]
    t = jnp.maximum(t, 0.0)
    o_ref[...] = dot(w3_ref[...], t, (((1,), (0,)), ((), ()))) + b3_ref[...]


_dense = pl.pallas_call(
    _tc_body,
    out_shape=jax.ShapeDtypeStruct((10, 64), jnp.float32),
)


def kernel(x, edge_index, edge_attr, W_l1, b_l1, W_r1, W_l2, b_l2, W_r2,
           W1, b1, W2, b2, W3, b3):
    src = edge_index[0].astype(jnp.int32)
    dst = edge_index[1].astype(jnp.int32)
    pad = E_PAD - src.shape[0]
    src_p = jnp.pad(src, (0, pad)).reshape(NW, EPT)
    dst_p = jnp.pad(dst, (0, pad)).reshape(NW, EPT)
    w_p = jnp.pad(edge_attr.astype(jnp.float32), (0, pad)).reshape(NW, EPT)

    a_parts = _get_build_adj()(src_p, dst_p, w_p).reshape(NC, NP, NP)

    x_pad = jnp.pad(x, ((0, NP - x.shape[0]), (0, 0)))
    w1_pad = jnp.pad(W1, ((0, 0), (0, NP - W1.shape[1])))
    out = _dense(a_parts, x_pad, W_l1, b_l1.reshape(1, -1), W_r1,
                 W_l2, b_l2.reshape(1, -1), W_r2,
                 w1_pad, b1.reshape(-1, 1), W2, b2.reshape(-1, 1),
                 W3, b3.reshape(-1, 1))
    return out.T


# speedup vs baseline: 1.2386x; 1.0370x over previous
import functools

import jax
import jax.numpy as jnp
from jax import lax
from jax.experimental import pallas as pl
from jax.experimental.pallas import tpu as pltpu
from jax.experimental.pallas import tpu_sc as plsc

EDGE_COUNT = 100000
NP = 256
ACC = 2 * NP * 128
NC, NS, LANES = 2, 16, 16
NW = NC * NS
EPT = 3200
ZPT = ACC // NS


def _sc_body(src_hbm, dst_hbm, ea_hbm, out_hbm,
             src_v, dst_v, w_v, one_v, idxw_v, idxc_v, buf_v, sem, acc_sh):
    cid = lax.axis_index("c")
    sid = lax.axis_index("s")
    tile = cid * NS + sid
    base = tile * EPT

    start = jnp.minimum(base, EDGE_COUNT - EPT)
    cp_s = pltpu.async_copy(src_hbm.at[pl.ds(start, EPT)], src_v, sem)
    cp_d = pltpu.async_copy(dst_hbm.at[pl.ds(start, EPT)], dst_v, sem)
    cp_w = pltpu.async_copy(ea_hbm.at[pl.ds(start, EPT)], w_v, sem)

    def zloop(i, c):
        buf_v[pl.ds(i * LANES, LANES)] = jnp.zeros((LANES,), jnp.float32)
        return c
    lax.fori_loop(0, ZPT // LANES, zloop, 0)
    pltpu.sync_copy(buf_v, acc_sh.at[pl.ds(sid * ZPT, ZPT)])

    cp_s.wait()
    cp_d.wait()
    cp_w.wait()

    def iloop(r, c):
        lane = lax.iota(jnp.int32, LANES)
        off = r * LANES
        valid = (start + off + lane) >= base
        s = jnp.where(valid, src_v[pl.ds(off, LANES)], 0)
        d = jnp.where(valid, dst_v[pl.ds(off, LANES)], 0)
        w_v[pl.ds(off, LANES)] = jnp.where(valid, w_v[pl.ds(off, LANES)], 0.0)
        idxw_v[pl.ds(off, LANES)] = \
            (s >> 7) * (NP * 128) + d * 128 + (s & 127)
        idxc_v[pl.ds(off, LANES)] = (NP * 128) + d * 128 + 72 + lane
        one_v[pl.ds(off, LANES)] = jnp.where(valid, 1.0, 0.0)
        return c
    lax.fori_loop(0, EPT // LANES, iloop, 0)

    plsc.subcore_barrier()

    pltpu.sync_copy(w_v, acc_sh.at[idxw_v], add=True)
    pltpu.sync_copy(one_v, acc_sh.at[idxc_v], add=True)

    plsc.subcore_barrier()

    pltpu.sync_copy(acc_sh.at[pl.ds(sid * ZPT, ZPT)], buf_v)
    pltpu.sync_copy(buf_v, out_hbm.at[cid, pl.ds(sid * ZPT, ZPT)])


@functools.cache
def _get_build_adj():
    return pl.kernel(
        _sc_body,
        out_type=jax.ShapeDtypeStruct((NC, ACC), jnp.float32),
        mesh=plsc.VectorSubcoreMesh(core_axis_name="c", subcore_axis_name="s",
                                    num_cores=NC, num_subcores=NS),
        scratch_types=[
            pltpu.VMEM((EPT,), jnp.int32),
            pltpu.VMEM((EPT,), jnp.int32),
            pltpu.VMEM((EPT,), jnp.float32),
            pltpu.VMEM((EPT,), jnp.float32),
            pltpu.VMEM((EPT,), jnp.int32),
            pltpu.VMEM((EPT,), jnp.int32),
            pltpu.VMEM((ZPT,), jnp.float32),
            pltpu.SemaphoreType.DMA,
            pltpu.VMEM_SHARED((ACC,), jnp.float32),
        ],
    )


def _tc_body(ap_ref, x_ref, wl1_ref, bl1_ref, wr1_ref, wl2_ref,
             bl2_ref, wr2_ref, w1_ref, b1_ref, w2_ref, b2_ref, w3_ref,
             b3_ref, o_ref):
    f32 = jnp.float32
    dot = functools.partial(lax.dot_general, preferred_element_type=f32)

    af = ap_ref[0:2 * NP] + ap_ref[2 * NP:4 * NP]
    a1 = af[0:NP]
    a2 = af[NP:2 * NP]
    lanes2 = lax.broadcasted_iota(jnp.int32, (NP, 128), 1)
    cnt = jnp.sum(jnp.where(lanes2 >= 72, a2, 0.0), axis=1, keepdims=True)
    inv = 1.0 / jnp.maximum(cnt, 1.0)
    rowmask = (lax.broadcasted_iota(jnp.int32, (NP, 1), 0) < 200).astype(f32)

    x = x_ref[...]
    mean1 = (dot(a1, x[0:128], (((1,), (0,)), ((), ())))
             + dot(a2, x[128:NP], (((1,), (0,)), ((), ())))) * inv
    h = dot(mean1, wl1_ref[...], (((1,), (1,)), ((), ()))) + bl1_ref[...] \
        + dot(x, wr1_ref[...], (((1,), (1,)), ((), ())))
    h = jnp.maximum(h, 0.0) * rowmask
    mean2 = (dot(a1, h[0:128], (((1,), (0,)), ((), ())))
             + dot(a2, h[128:NP], (((1,), (0,)), ((), ())))) * inv
    out = dot(mean2, wl2_ref[...], (((1,), (1,)), ((), ()))) + bl2_ref[...] \
        + dot(h, wr2_ref[...], (((1,), (1,)), ((), ())))
    t = dot(w1_ref[...], out, (((1,), (0,)), ((), ()))) + b1_ref[...]
    t = jnp.maximum(t, 0.0)
    t = dot(w2_ref[...], t, (((1,), (0,)), ((), ()))) + b2_ref[...]
    t = jnp.maximum(t, 0.0)
    o_ref[...] = dot(w3_ref[...], t, (((1,), (0,)), ((), ()))) + b3_ref[...]


_dense = pl.pallas_call(
    _tc_body,
    out_shape=jax.ShapeDtypeStruct((10, 64), jnp.float32),
)


def kernel(x, edge_index, edge_attr, W_l1, b_l1, W_r1, W_l2, b_l2, W_r2,
           W1, b1, W2, b2, W3, b3):
    ei = edge_index.astype(jnp.int32)
    ea = edge_attr.astype(jnp.float32)

    a_parts = _get_build_adj()(ei[0], ei[1], ea).reshape(NC * 2 * NP, 128)

    x_pad = jnp.pad(x, ((0, NP - x.shape[0]), (0, 0)))
    w1_pad = jnp.pad(W1, ((0, 0), (0, NP - W1.shape[1])))
    out = _dense(a_parts, x_pad, W_l1, b_l1.reshape(1, -1), W_r1,
                 W_l2, b_l2.reshape(1, -1), W_r2,
                 w1_pad, b1.reshape(-1, 1), W2, b2.reshape(-1, 1),
                 W3, b3.reshape(-1, 1))
    return out.T


# speedup vs baseline: 1.3472x; 1.0877x over previous
import functools

import jax
import jax.numpy as jnp
from jax import lax
from jax.experimental import pallas as pl
from jax.experimental.pallas import tpu as pltpu
from jax.experimental.pallas import tpu_sc as plsc

EDGE_COUNT = 100000
NP = 256
ACC = 2 * NP * 128
NC, NS, LANES = 2, 16, 16
NW = NC * NS
EPT = 3200
ZPT = ACC // NS


def _sc_body(src_hbm, dst_hbm, ea_hbm, out_hbm,
             src_v, dst_v, w_v, one_v, idxw_v, idxc_v, buf_v, sem, sem2,
             acc_sh):
    cid = lax.axis_index("c")
    sid = lax.axis_index("s")
    tile = cid * NS + sid
    base = tile * EPT

    start = jnp.minimum(base, EDGE_COUNT - EPT)
    cp_s = pltpu.async_copy(src_hbm.at[pl.ds(start, EPT)], src_v, sem)
    cp_d = pltpu.async_copy(dst_hbm.at[pl.ds(start, EPT)], dst_v, sem)
    cp_w = pltpu.async_copy(ea_hbm.at[pl.ds(start, EPT)], w_v, sem)

    def zloop(i, c):
        buf_v[pl.ds(i * LANES, LANES)] = jnp.zeros((LANES,), jnp.float32)
        return c
    lax.fori_loop(0, ZPT // LANES, zloop, 0)
    pltpu.sync_copy(buf_v, acc_sh.at[pl.ds(sid * ZPT, ZPT)])

    cp_s.wait()
    cp_d.wait()
    cp_w.wait()

    def iloop(r, c):
        lane = lax.iota(jnp.int32, LANES)
        off = r * LANES
        valid = (start + off + lane) >= base
        s = jnp.where(valid, src_v[pl.ds(off, LANES)], 0)
        d = jnp.where(valid, dst_v[pl.ds(off, LANES)], 0)
        w_v[pl.ds(off, LANES)] = jnp.where(valid, w_v[pl.ds(off, LANES)], 0.0)
        idxw_v[pl.ds(off, LANES)] = \
            (s >> 7) * (NP * 128) + d * 128 + (s & 127)
        idxc_v[pl.ds(off, LANES)] = (NP * 128) + d * 128 + 72 + lane
        one_v[pl.ds(off, LANES)] = jnp.where(valid, 1.0, 0.0)
        return c
    lax.fori_loop(0, EPT // LANES, iloop, 0)

    plsc.subcore_barrier()

    cp_a = pltpu.async_copy(w_v, acc_sh.at[idxw_v], sem, add=True)
    cp_b = pltpu.async_copy(one_v, acc_sh.at[idxc_v], sem2, add=True)
    cp_a.wait()
    cp_b.wait()

    plsc.subcore_barrier()

    pltpu.sync_copy(acc_sh.at[pl.ds(sid * ZPT, ZPT)], buf_v)
    pltpu.sync_copy(buf_v, out_hbm.at[pl.ds(cid * ACC + sid * ZPT, ZPT)])


@functools.cache
def _get_build_adj():
    return pl.kernel(
        _sc_body,
        out_type=jax.ShapeDtypeStruct((NC * ACC,), jnp.float32),
        mesh=plsc.VectorSubcoreMesh(core_axis_name="c", subcore_axis_name="s",
                                    num_cores=NC, num_subcores=NS),
        scratch_types=[
            pltpu.VMEM((EPT,), jnp.int32),
            pltpu.VMEM((EPT,), jnp.int32),
            pltpu.VMEM((EPT,), jnp.float32),
            pltpu.VMEM((EPT,), jnp.float32),
            pltpu.VMEM((EPT,), jnp.int32),
            pltpu.VMEM((EPT,), jnp.int32),
            pltpu.VMEM((ZPT,), jnp.float32),
            pltpu.SemaphoreType.DMA,
            pltpu.SemaphoreType.DMA,
            pltpu.VMEM_SHARED((ACC,), jnp.float32),
        ],
    )


def _tc_body(ap_ref, x_ref, wl1_ref, bl1_ref, wr1_ref, wl2_ref,
             bl2_ref, wr2_ref, w1_ref, b1_ref, w2_ref, b2_ref, w3_ref,
             b3_ref, o_ref):
    f32 = jnp.float32
    dot = functools.partial(lax.dot_general, preferred_element_type=f32)

    af = ap_ref[0:2 * NP] + ap_ref[2 * NP:4 * NP]
    a1 = af[0:NP]
    a2 = af[NP:2 * NP]
    lanes2 = lax.broadcasted_iota(jnp.int32, (NP, 128), 1)
    cnt = jnp.sum(jnp.where(lanes2 >= 72, a2, 0.0), axis=1, keepdims=True)
    inv = 1.0 / jnp.maximum(cnt, 1.0)
    rowmask = (lax.broadcasted_iota(jnp.int32, (NP, 1), 0) < 200).astype(f32)

    x = x_ref[...]
    mean1 = (dot(a1, x[0:128], (((1,), (0,)), ((), ())))
             + dot(a2, x[128:NP], (((1,), (0,)), ((), ())))) * inv
    h = dot(mean1, wl1_ref[...], (((1,), (1,)), ((), ()))) + bl1_ref[...] \
        + dot(x, wr1_ref[...], (((1,), (1,)), ((), ())))
    h = jnp.maximum(h, 0.0) * rowmask
    mean2 = (dot(a1, h[0:128], (((1,), (0,)), ((), ())))
             + dot(a2, h[128:NP], (((1,), (0,)), ((), ())))) * inv
    out = dot(mean2, wl2_ref[...], (((1,), (1,)), ((), ()))) + bl2_ref[...] \
        + dot(h, wr2_ref[...], (((1,), (1,)), ((), ())))
    t = dot(w1_ref[...], out, (((1,), (0,)), ((), ()))) + b1_ref[...]
    t = jnp.maximum(t, 0.0)
    t = dot(w2_ref[...], t, (((1,), (0,)), ((), ()))) + b2_ref[...]
    t = jnp.maximum(t, 0.0)
    o_ref[...] = dot(w3_ref[...], t, (((1,), (0,)), ((), ()))) + b3_ref[...]


_dense = pl.pallas_call(
    _tc_body,
    out_shape=jax.ShapeDtypeStruct((10, 64), jnp.float32),
)


def kernel(x, edge_index, edge_attr, W_l1, b_l1, W_r1, W_l2, b_l2, W_r2,
           W1, b1, W2, b2, W3, b3):
    ei = edge_index.astype(jnp.int32)
    ea = edge_attr.astype(jnp.float32)

    a_parts = _get_build_adj()(ei[0], ei[1], ea).reshape(NC * 2 * NP, 128)

    x_pad = jnp.pad(x, ((0, NP - x.shape[0]), (0, 0)))
    w1_pad = jnp.pad(W1, ((0, 0), (0, NP - W1.shape[1])))
    out = _dense(a_parts, x_pad, W_l1, b_l1.reshape(1, -1), W_r1,
                 W_l2, b_l2.reshape(1, -1), W_r2,
                 w1_pad, b1.reshape(-1, 1), W2, b2.reshape(-1, 1),
                 W3, b3.reshape(-1, 1))
    return out.T


# speedup vs baseline: 1.3542x; 1.0052x over previous
import functools

import jax
import jax.numpy as jnp
from jax import lax
from jax.experimental import pallas as pl
from jax.experimental.pallas import tpu as pltpu
from jax.experimental.pallas import tpu_sc as plsc

EDGE_COUNT = 100000
NP = 256
ACC = 2 * NP * 128
NC, NS, LANES = 2, 16, 16
NW = NC * NS
EPT = 3200
ZPT = ACC // NS


def _sc_body(src_hbm, dst_hbm, ea_hbm, out_hbm,
             src_v, dst_v, w_v, one_v, idxw_v, idxc_v, buf_v, sem, sem2,
             acc_sh):
    cid = lax.axis_index("c")
    sid = lax.axis_index("s")
    tile = cid * NS + sid
    base = tile * EPT

    start = jnp.minimum(base, EDGE_COUNT - EPT)
    cp_s = pltpu.async_copy(src_hbm.at[pl.ds(start, EPT)], src_v, sem)
    cp_d = pltpu.async_copy(dst_hbm.at[pl.ds(start, EPT)], dst_v, sem)
    cp_w = pltpu.async_copy(ea_hbm.at[pl.ds(start, EPT)], w_v, sem)

    def zloop(i, c):
        buf_v[pl.ds(i * LANES, LANES)] = jnp.zeros((LANES,), jnp.float32)
        return c
    lax.fori_loop(0, ZPT // LANES, zloop, 0)
    pltpu.sync_copy(buf_v, acc_sh.at[pl.ds(sid * ZPT, ZPT)])

    cp_s.wait()
    cp_d.wait()
    cp_w.wait()

    def iloop(r, c):
        lane = lax.iota(jnp.int32, LANES)
        off = r * LANES
        valid = (start + off + lane) >= base
        s = jnp.where(valid, src_v[pl.ds(off, LANES)], 0)
        d = jnp.where(valid, dst_v[pl.ds(off, LANES)], 0)
        w_v[pl.ds(off, LANES)] = jnp.where(valid, w_v[pl.ds(off, LANES)], 0.0)
        idxw_v[pl.ds(off, LANES)] = \
            (s >> 7) * (NP * 128) + d * 128 + (s & 127)
        idxc_v[pl.ds(off, LANES)] = (NP * 128) + d * 128 + 72 + lane
        one_v[pl.ds(off, LANES)] = jnp.where(valid, 1.0, 0.0)
        return c
    lax.fori_loop(0, EPT // LANES, iloop, 0)

    plsc.subcore_barrier()

    cp_a = pltpu.async_copy(w_v, acc_sh.at[idxw_v], sem, add=True)
    cp_b = pltpu.async_copy(one_v, acc_sh.at[idxc_v], sem2, add=True)
    cp_a.wait()
    cp_b.wait()

    plsc.subcore_barrier()

    pltpu.sync_copy(acc_sh.at[pl.ds(sid * ZPT, ZPT)], buf_v)
    pltpu.sync_copy(buf_v, out_hbm.at[pl.ds(cid * ACC + sid * ZPT, ZPT)])


@functools.cache
def _get_build_adj():
    return pl.kernel(
        _sc_body,
        out_type=jax.ShapeDtypeStruct((NC * ACC,), jnp.float32),
        mesh=plsc.VectorSubcoreMesh(core_axis_name="c", subcore_axis_name="s",
                                    num_cores=NC, num_subcores=NS),
        scratch_types=[
            pltpu.VMEM((EPT,), jnp.int32),
            pltpu.VMEM((EPT,), jnp.int32),
            pltpu.VMEM((EPT,), jnp.float32),
            pltpu.VMEM((EPT,), jnp.float32),
            pltpu.VMEM((EPT,), jnp.int32),
            pltpu.VMEM((EPT,), jnp.int32),
            pltpu.VMEM((ZPT,), jnp.float32),
            pltpu.SemaphoreType.DMA,
            pltpu.SemaphoreType.DMA,
            pltpu.VMEM_SHARED((ACC,), jnp.float32),
        ],
    )


def _tc_body(ap_ref, x_ref, wl1_ref, bl1_ref, wr1_ref, wl2_ref,
             bl2_ref, wr2_ref, w1_ref, b1_ref, w2_ref, b2_ref, w3_ref,
             b3_ref, o_ref):
    f32 = jnp.float32
    dot = functools.partial(lax.dot_general, preferred_element_type=f32)

    af = ap_ref[0:2 * NP] + ap_ref[2 * NP:4 * NP]
    a1 = af[0:NP]
    a2 = af[NP:2 * NP]
    lanes2 = lax.broadcasted_iota(jnp.int32, (NP, 128), 1)
    cnt = jnp.sum(jnp.where(lanes2 >= 72, a2, 0.0), axis=1, keepdims=True)
    inv = 1.0 / jnp.maximum(cnt, 1.0)
    rowmask = (lax.broadcasted_iota(jnp.int32, (NP, 1), 0) < 200).astype(f32)

    x = x_ref[...]
    bf = jnp.bfloat16
    a1b, a2b, xb = a1.astype(bf), a2.astype(bf), x.astype(bf)
    mean1 = (dot(a1b, xb[0:128], (((1,), (0,)), ((), ())))
             + dot(a2b, xb[128:NP], (((1,), (0,)), ((), ())))) * inv
    h = dot(mean1.astype(bf), wl1_ref[...], (((1,), (1,)), ((), ()))) \
        + bl1_ref[...] \
        + dot(xb, wr1_ref[...], (((1,), (1,)), ((), ())))
    h = jnp.maximum(h, 0.0) * rowmask
    mean2 = (dot(a1, h[0:128], (((1,), (0,)), ((), ())))
             + dot(a2, h[128:NP], (((1,), (0,)), ((), ())))) * inv
    out = dot(mean2, wl2_ref[...], (((1,), (1,)), ((), ()))) + bl2_ref[...] \
        + dot(h, wr2_ref[...], (((1,), (1,)), ((), ())))
    t = dot(w1_ref[...], out, (((1,), (0,)), ((), ()))) + b1_ref[...]
    t = jnp.maximum(t, 0.0)
    t = dot(w2_ref[...], t, (((1,), (0,)), ((), ()))) + b2_ref[...]
    t = jnp.maximum(t, 0.0)
    o_ref[...] = dot(w3_ref[...], t, (((1,), (0,)), ((), ()))) + b3_ref[...]


_dense = pl.pallas_call(
    _tc_body,
    out_shape=jax.ShapeDtypeStruct((10, 64), jnp.float32),
)


def kernel(x, edge_index, edge_attr, W_l1, b_l1, W_r1, W_l2, b_l2, W_r2,
           W1, b1, W2, b2, W3, b3):
    ei = edge_index.astype(jnp.int32)
    ea = edge_attr.astype(jnp.float32)

    a_parts = _get_build_adj()(ei[0], ei[1], ea).reshape(NC * 2 * NP, 128)

    x_pad = jnp.pad(x, ((0, NP - x.shape[0]), (0, 0)))
    w1_pad = jnp.pad(W1, ((0, 0), (0, NP - W1.shape[1])))
    out = _dense(a_parts, x_pad, W_l1.astype(jnp.bfloat16),
                 b_l1.reshape(1, -1), W_r1.astype(jnp.bfloat16),
                 W_l2, b_l2.reshape(1, -1), W_r2,
                 w1_pad, b1.reshape(-1, 1), W2, b2.reshape(-1, 1),
                 W3, b3.reshape(-1, 1))
    return out.T
